# Initial kernel scaffold; baseline (speedup 1.0000x reference)
import jax
import jax.numpy as jnp
from jax.experimental import pallas as pl


def kernel(x, edge_index, edge_attr, neg_edge_index, g1_lin_w, g1_att_src, g1_att_dst, g1_lin_edge_w, g1_att_edge, g1_bias, g2_lin_w, g2_att_src, g2_att_dst, g2_lin_edge_w, g2_att_edge, g2_bias, d1_w1, d1_b1, d1_w2, d1_b2, d2_w1, d2_b1, d2_w2, d2_b2):
    raise NotImplementedError("write your pallas kernel here")


# speedup vs baseline: 26.1511x; 26.1511x over previous
import functools

import jax
import jax.numpy as jnp
from jax import lax
from jax.experimental import pallas as pl
from jax.experimental.pallas import tpu as pltpu
from jax.experimental.pallas import tpu_sc as plsc

N = 10000
E = 320000
D = 128
DE = 16
H = 3
HC = 128
F = H * HC
EL = E + N
EP = 330240
NPAD = 10240
DENF = H * NPAD
NCORES = 2
NSUB = 16
NW = NCORES * NSUB
EPW = EP // NW
SUB = 80
ROWS_ALL = EP // SUB
ROWS_W = ROWS_ALL // NW
KA = 688
NCHA = EPW // KA
NSUPB = 3
SUBS_B = ROWS_W // NSUPB
EDGE_B = SUBS_B * SUB
HALFC = 192
EPT_C = E // NW
ROWS_C = EPT_C // SUB
NSUPC = 5
SUBS_C = ROWS_C // NSUPC
EDGE_C = SUBS_C * SUB
NSTRIPE = N // NSUB
ZROWS = 25


def _f32(*shape):
    return jax.ShapeDtypeStruct(shape, jnp.float32)


NB = 400
NBLKS = N // NB


def _proj1_body(x_ref, w_ref, am_ref, xp_ref, a_ref):
    xb = jnp.nan_to_num(x_ref[...], nan=0.0)
    xp = lax.dot_general(xb, w_ref[...], (((1,), (1,)), ((), ())),
                         preferred_element_type=jnp.float32)
    xp_ref[...] = xp
    a_ref[...] = lax.dot_general(xp, am_ref[...], (((1,), (1,)), ((), ())),
                                 preferred_element_type=jnp.float32)


def _tc_proj1(x, w, att_mat):
    return pl.pallas_call(
        _proj1_body,
        grid=(NBLKS,),
        in_specs=[
            pl.BlockSpec((NB, D), lambda i: (i, 0)),
            pl.BlockSpec((F, D), lambda i: (0, 0)),
            pl.BlockSpec((8, F), lambda i: (0, 0)),
        ],
        out_specs=[
            pl.BlockSpec((NB, F), lambda i: (i, 0)),
            pl.BlockSpec((NB, 8), lambda i: (i, 0)),
        ],
        out_shape=[_f32(N, F), _f32(N, 8)],
    )(x, w, att_mat)


EB = 1280
EBLKS = EP // EB


def _edge_att_body(ea_ref, lw1_ref, lw2_ref, m1_ref, m2_ref, o_ref):
    w1t = lax.dot_general(m1_ref[...], lw1_ref[...], (((1,), (0,)), ((), ())),
                          preferred_element_type=jnp.float32)
    w2t = lax.dot_general(m2_ref[...], lw2_ref[...], (((1,), (0,)), ((), ())),
                          preferred_element_type=jnp.float32)
    wcat = jnp.concatenate([w1t, w2t], axis=0)
    ae = lax.dot_general(wcat, ea_ref[...], (((1,), (1,)), ((), ())),
                         preferred_element_type=jnp.float32)
    i = pl.program_id(0)
    col = i * EB + lax.broadcasted_iota(jnp.int32, (6, EB), 1)
    o_ref[...] = jnp.where(col >= EL, -1e30, ae)


def _tc_edge_att(ea_pad, lw1, lw2, ae_mat1, ae_mat2):
    return pl.pallas_call(
        _edge_att_body,
        grid=(EBLKS,),
        in_specs=[
            pl.BlockSpec((EB, DE), lambda i: (i, 0)),
            pl.BlockSpec((F, DE), lambda i: (0, 0)),
            pl.BlockSpec((F, DE), lambda i: (0, 0)),
            pl.BlockSpec((3, F), lambda i: (0, 0)),
            pl.BlockSpec((3, F), lambda i: (0, 0)),
        ],
        out_specs=pl.BlockSpec((6, EB), lambda i: (0, i)),
        out_shape=_f32(6, EP),
    )(ea_pad, lw1, lw2, ae_mat1, ae_mat2)


def _proj2_body(aa_ref, ab_ref, ac_ref, b_ref, w_ref, am_ref, xp_ref, a_ref):
    agg = jnp.concatenate(
        [aa_ref[0] + aa_ref[1], ab_ref[0] + ab_ref[1],
         ac_ref[0] + ac_ref[1]], axis=1) + b_ref[...]
    h = jnp.where(agg > 0, agg, jnp.exp(jnp.minimum(agg, 0.0)) - 1.0)
    xp = lax.dot_general(h, w_ref[...], (((1,), (1,)), ((), ())),
                         preferred_element_type=jnp.float32)
    xp_ref[...] = xp
    a_ref[...] = lax.dot_general(xp, am_ref[...], (((1,), (1,)), ((), ())),
                                 preferred_element_type=jnp.float32)


def _tc_proj2(aggs, bias, w, att_mat):
    return pl.pallas_call(
        _proj2_body,
        grid=(NBLKS,),
        in_specs=[
            pl.BlockSpec((2, NB, HC), lambda i: (0, i, 0)),
            pl.BlockSpec((2, NB, HC), lambda i: (0, i, 0)),
            pl.BlockSpec((2, NB, HC), lambda i: (0, i, 0)),
            pl.BlockSpec((1, F), lambda i: (0, 0)),
            pl.BlockSpec((F, F), lambda i: (0, 0)),
            pl.BlockSpec((8, F), lambda i: (0, 0)),
        ],
        out_specs=[
            pl.BlockSpec((NB, F), lambda i: (i, 0)),
            pl.BlockSpec((NB, 8), lambda i: (i, 0)),
        ],
        out_shape=[_f32(N, F), _f32(N, 8)],
    )(aggs[0], aggs[1], aggs[2], bias, w, att_mat)


def _head_body(aa_ref, ab_ref, ac_ref, b_ref, w1_ref, b1_ref, w2_ref, b2_ref,
               wl_ref, wr_ref, z_ref, lg_ref, u_ref, v_ref):
    z = jnp.concatenate(
        [aa_ref[0] + aa_ref[1], ab_ref[0] + ab_ref[1],
         ac_ref[0] + ac_ref[1]], axis=1) + b_ref[...]
    z_ref[...] = z
    t = lax.dot_general(z, w1_ref[...], (((1,), (1,)), ((), ())),
                        preferred_element_type=jnp.float32) + b1_ref[...]
    t = jnp.maximum(t, 0.0)
    lg = lax.dot_general(t, w2_ref[...], (((1,), (1,)), ((), ())),
                         preferred_element_type=jnp.float32) + b2_ref[...]
    lg_ref[...] = jax.nn.sigmoid(lg)
    u_ref[...] = lax.dot_general(z, wl_ref[...], (((1,), (1,)), ((), ())),
                                 preferred_element_type=jnp.float32)
    v_ref[...] = lax.dot_general(z, wr_ref[...], (((1,), (1,)), ((), ())),
                                 preferred_element_type=jnp.float32)


def _tc_head(aggs, bias, d2w1, d2b1, d2w2, d2b2, wl, wr):
    return pl.pallas_call(
        _head_body,
        grid=(NBLKS,),
        in_specs=[
            pl.BlockSpec((2, NB, HC), lambda i: (0, i, 0)),
            pl.BlockSpec((2, NB, HC), lambda i: (0, i, 0)),
            pl.BlockSpec((2, NB, HC), lambda i: (0, i, 0)),
            pl.BlockSpec((1, F), lambda i: (0, 0)),
            pl.BlockSpec((64, F), lambda i: (0, 0)),
            pl.BlockSpec((1, 64), lambda i: (0, 0)),
            pl.BlockSpec((16, 64), lambda i: (0, 0)),
            pl.BlockSpec((1, 16), lambda i: (0, 0)),
            pl.BlockSpec((64, F), lambda i: (0, 0)),
            pl.BlockSpec((64, F), lambda i: (0, 0)),
        ],
        out_specs=[
            pl.BlockSpec((NB, F), lambda i: (i, 0)),
            pl.BlockSpec((NB, 16), lambda i: (i, 0)),
            pl.BlockSpec((NB, 64), lambda i: (i, 0)),
            pl.BlockSpec((NB, 64), lambda i: (i, 0)),
        ],
        out_shape=[_f32(N, F), _f32(N, 16), _f32(N, 64), _f32(N, 64)],
    )(aggs[0], aggs[1], aggs[2], bias, d2w1, d2b1, d2w2, d2b2, wl, wr)


_MESH = plsc.VectorSubcoreMesh(core_axis_name="c", subcore_axis_name="s")
_SC_PARAMS = pltpu.CompilerParams(use_tc_tiling_on_sc=False,
                                  needs_layout_passes=False)


def _sc_alpha_body(src_hbm, dst_hbm, ae_hbm, atab_hbm, ex_hbm, den_hbm,
                   a_loc, den_loc, srcb, dstb, aeb, exb):
    c = lax.axis_index("c")
    s = lax.axis_index("s")
    wid = s * NCORES + c

    pltpu.sync_copy(atab_hbm, a_loc)

    def zero_body(j, _):
        den_loc[pl.ds(j * 16, 16)] = jnp.zeros((16,), jnp.float32)
        return 0
    lax.fori_loop(0, DENF // 16, zero_body, 0)

    def chunk_body(j, _):
        base = pl.multiple_of(wid * EPW + j * KA, 16)
        pltpu.sync_copy(src_hbm.at[pl.ds(base, KA)], srcb)
        pltpu.sync_copy(dst_hbm.at[pl.ds(base, KA)], dstb)
        for h in range(H):
            pltpu.sync_copy(ae_hbm.at[pl.ds(h * EP + base, KA)], aeb.at[h])

        def grp_body(g, _):
            off = g * 16
            si = srcb[pl.ds(off, 16)]
            di = dstb[pl.ds(off, 16)]
            for h in range(H):
                asv = plsc.load_gather(a_loc, [si + h * N])
                adv = plsc.load_gather(a_loc, [di + (H + h) * N])
                al = asv + adv + aeb[h, pl.ds(off, 16)]
                al = jnp.where(al >= 0, al, al * 0.2)
                ev = jnp.exp(al)
                exb[h, pl.ds(off, 16)] = ev
                plsc.addupdate_scatter(den_loc, [di + h * NPAD], ev)
            return 0
        lax.fori_loop(0, KA // 16, grp_body, 0)

        for h in range(H):
            pltpu.sync_copy(exb.at[h], ex_hbm.at[pl.ds(h * EP + base, KA)])
        return 0
    lax.fori_loop(0, NCHA, chunk_body, 0)

    pltpu.sync_copy(den_loc, den_hbm.at[pl.ds(wid * DENF, DENF)])


def _sc_alpha(src2, dst2, aeT, atab):
    kfn = pl.kernel(
        _sc_alpha_body,
        out_type=(_f32(H * EP), _f32(NW * DENF)),
        mesh=_MESH,
        compiler_params=_SC_PARAMS,
        scratch_types=[
            pltpu.VMEM((8 * N,), jnp.float32),
            pltpu.VMEM((DENF,), jnp.float32),
            pltpu.VMEM((KA,), jnp.int32),
            pltpu.VMEM((KA,), jnp.int32),
            pltpu.VMEM((H, KA), jnp.float32),
            pltpu.VMEM((H, KA), jnp.float32),
        ],
    )
    src_flat = src2.reshape(EP)
    dst_flat = dst2.reshape(EP)
    return kfn(src_flat, dst_flat, aeT, atab)


DBLK = 3840


def _den_red_body(p_ref, o_ref):
    o_ref[...] = jnp.sum(p_ref[...], axis=0, keepdims=True)


def _tc_den_reduce(denp):
    return pl.pallas_call(
        _den_red_body,
        grid=(DENF // DBLK,),
        in_specs=[pl.BlockSpec((NW, DBLK), lambda j: (0, j))],
        out_specs=pl.BlockSpec((1, DBLK), lambda j: (0, j)),
        out_shape=_f32(1, DENF),
    )(denp.reshape(NW, DENF))


def _sc_msg_body(src2_hbm, dst2_hbm, ex_hbm, den_hbm, xph_hbm, out_hbm,
                 den_loc, srcb2, dstb2, exb, coefb, rows, zbuf, acc, sem):
    c = lax.axis_index("c")
    s = lax.axis_index("s")
    wid = s * NCORES + c

    pltpu.sync_copy(den_hbm, den_loc)

    def zzero(j, _):
        def zrow(g, _):
            zbuf[j, pl.ds(g * 16, 16)] = jnp.zeros((16,), jnp.float32)
            return 0
        lax.fori_loop(0, HC // 16, zrow, 0)
        return 0
    lax.fori_loop(0, ZROWS, zzero, 0)

    def zcopy(k, _):
        pltpu.sync_copy(
            zbuf, acc.at[pl.ds(s * NSTRIPE + k * ZROWS, ZROWS), :])
        return 0
    lax.fori_loop(0, NSTRIPE // ZROWS, zcopy, 0)
    plsc.subcore_barrier()

    for sup in range(NSUPB):
        rowbase = wid * ROWS_W + sup * SUBS_B
        ebase = rowbase * SUB
        pltpu.sync_copy(src2_hbm.at[pl.ds(rowbase, SUBS_B), :], srcb2)
        pltpu.sync_copy(dst2_hbm.at[pl.ds(rowbase, SUBS_B), :], dstb2)
        pltpu.sync_copy(ex_hbm.at[pl.ds(ebase, EDGE_B)], exb)

        def coef_body(g, _):
            sub = g // (SUB // 16)
            off = (g % (SUB // 16)) * 16
            di = dstb2[sub, pl.ds(off, 16)]
            fl = pl.ds(sub * SUB + off, 16)
            denv = plsc.load_gather(den_loc, [di])
            coefb[fl] = exb[fl] / denv
            return 0
        lax.fori_loop(0, EDGE_B // 16, coef_body, 0)

        def sub_body(k, _):
            pltpu.async_copy(xph_hbm.at[srcb2.at[k]], rows, sem).wait()

            def scale_body(e, _):
                csv = plsc.load_gather(
                    coefb, [jnp.full((16,), k * SUB + e, jnp.int32)])
                for grp in range(HC // 16):
                    sl = pl.ds(grp * 16, 16)
                    rows[e, sl] = rows[e, sl] * csv
                return 0
            lax.fori_loop(0, SUB, scale_body, 0)
            pltpu.sync_copy(rows, acc.at[dstb2.at[k]], add=True)
            return 0
        lax.fori_loop(0, SUBS_B, sub_body, 0)

    plsc.subcore_barrier()
    ro = s * NSTRIPE
    pltpu.sync_copy(acc.at[pl.ds(ro, NSTRIPE), :],
                    out_hbm.at[c, pl.ds(ro, NSTRIPE), :])


def _sc_msg(src2, dst2, exh, denh, xph):
    kfn = pl.kernel(
        _sc_msg_body,
        out_type=_f32(NCORES, N, HC),
        mesh=_MESH,
        compiler_params=_SC_PARAMS,
        scratch_types=[
            pltpu.VMEM((NPAD,), jnp.float32),
            pltpu.VMEM((SUBS_B, SUB), jnp.int32),
            pltpu.VMEM((SUBS_B, SUB), jnp.int32),
            pltpu.VMEM((EDGE_B,), jnp.float32),
            pltpu.VMEM((EDGE_B,), jnp.float32),
            pltpu.VMEM((SUB, HC), jnp.float32),
            pltpu.VMEM((ZROWS, HC), jnp.float32),
            pltpu.VMEM_SHARED((N, HC), jnp.float32),
            pltpu.SemaphoreType.DMA,
        ],
    )
    return kfn(src2, dst2, exh, denh, xph)


def _sc_decoder_body(ps2_hbm, pd2_hbm, ns2_hbm, nd2_hbm, u_hbm, v_hbm,
                     cst_hbm, pos_hbm, neg_hbm,
                     srcb2, dstb2, urows, vrows, predb, cstb, sem):
    c = lax.axis_index("c")
    s = lax.axis_index("s")
    wid = s * NCORES + c
    pltpu.sync_copy(cst_hbm, cstb)
    b2s = cstb[2, pl.ds(0, 16)][0]

    for (s2, d2, o_hbm) in ((ps2_hbm, pd2_hbm, pos_hbm),
                            (ns2_hbm, nd2_hbm, neg_hbm)):
        for sup in range(NSUPC):
            rowbase = wid * ROWS_C + sup * SUBS_C
            ebase = rowbase * SUB
            pltpu.sync_copy(s2.at[pl.ds(rowbase, SUBS_C), :], srcb2)
            pltpu.sync_copy(d2.at[pl.ds(rowbase, SUBS_C), :], dstb2)

            def sub_body(k, _):
                pltpu.async_copy(u_hbm.at[srcb2.at[k]], urows, sem).wait()
                pltpu.async_copy(v_hbm.at[dstb2.at[k]], vrows, sem).wait()
                lanes = lax.iota(jnp.int32, 16)

                def grp_body(g, _):
                    def edge_body(i, resv):
                        e = g * 16 + i
                        accv = jnp.zeros((16,), jnp.float32)
                        for grp in range(4):
                            sl = pl.ds(grp * 16, 16)
                            t = urows[e, sl] + vrows[e, sl] + cstb[0, sl]
                            t = jnp.maximum(t, 0.0)
                            accv = accv + t * cstb[1, sl]
                        tot = jnp.sum(accv) + b2s
                        return jnp.where(lanes == i, tot, resv)
                    resv = lax.fori_loop(
                        0, 16, edge_body, jnp.zeros((16,), jnp.float32))
                    predb[pl.ds(k * SUB + g * 16, 16)] = resv
                    return 0
                lax.fori_loop(0, SUB // 16, grp_body, 0)
                return 0
            lax.fori_loop(0, SUBS_C, sub_body, 0)
            pltpu.sync_copy(predb, o_hbm.at[pl.ds(ebase, EDGE_C)])


def _sc_decoder(ps2, pd2, ns2, nd2, u, v, cst):
    kfn = pl.kernel(
        _sc_decoder_body,
        out_type=(_f32(E), _f32(E)),
        mesh=_MESH,
        compiler_params=_SC_PARAMS,
        scratch_types=[
            pltpu.VMEM((SUBS_C, SUB), jnp.int32),
            pltpu.VMEM((SUBS_C, SUB), jnp.int32),
            pltpu.VMEM((SUB, 64), jnp.float32),
            pltpu.VMEM((SUB, 64), jnp.float32),
            pltpu.VMEM((EDGE_C,), jnp.float32),
            pltpu.VMEM((3, 64), jnp.float32),
            pltpu.SemaphoreType.DMA,
        ],
    )
    return kfn(ps2, pd2, ns2, nd2, u, v, cst)


def _place_heads(att):
    rows = []
    for h in range(H):
        parts = [jnp.zeros((HC,), jnp.float32)] * H
        parts[h] = att[0, h]
        rows.append(jnp.concatenate(parts))
    return jnp.stack(rows)


def _att_mat(att_src, att_dst):
    return jnp.concatenate(
        [_place_heads(att_src), _place_heads(att_dst),
         jnp.zeros((2, F), jnp.float32)], axis=0)


def _gat_layer(xp, a_tab, src2, dst2, aeT3):
    exT, denp = _sc_alpha(src2, dst2, aeT3, a_tab.T.reshape(8 * N))
    den = _tc_den_reduce(denp).reshape(DENF)
    xp3 = xp.reshape(N, H, HC).transpose(1, 0, 2)
    return tuple(
        _sc_msg(src2, dst2, exT[h * EP:(h + 1) * EP],
                den[h * NPAD:(h + 1) * NPAD], xp3[h])
        for h in range(H))


def kernel(x, edge_index, edge_attr, neg_edge_index, g1_lin_w, g1_att_src,
           g1_att_dst, g1_lin_edge_w, g1_att_edge, g1_bias, g2_lin_w,
           g2_att_src, g2_att_dst, g2_lin_edge_w, g2_att_edge, g2_bias,
           d1_w1, d1_b1, d1_w2, d1_b2, d2_w1, d2_b1, d2_w2, d2_b2):
    loop = jnp.arange(N, dtype=jnp.int32)
    padi = jnp.zeros((EP - EL,), jnp.int32)
    src2 = jnp.concatenate([edge_index[0], loop, padi]).reshape(ROWS_ALL, SUB)
    dst2 = jnp.concatenate([edge_index[1], loop, padi]).reshape(ROWS_ALL, SUB)
    ea_pad = jnp.concatenate([
        edge_attr, jnp.ones((N, DE), jnp.float32),
        jnp.zeros((EP - EL, DE), jnp.float32)], axis=0)

    att_mat1 = _att_mat(g1_att_src, g1_att_dst)
    att_mat2 = _att_mat(g2_att_src, g2_att_dst)
    ae_mat1 = _place_heads(g1_att_edge)
    ae_mat2 = _place_heads(g2_att_edge)

    xp1, a1 = _tc_proj1(x, g1_lin_w, att_mat1)
    aeT = _tc_edge_att(ea_pad, g1_lin_edge_w, g2_lin_edge_w, ae_mat1, ae_mat2)

    aggs1 = _gat_layer(xp1, a1, src2, dst2, aeT[0:H].reshape(H * EP))

    xp2, a2 = _tc_proj2(aggs1, g1_bias.reshape(1, F), g2_lin_w, att_mat2)

    aggs2 = _gat_layer(xp2, a2, src2, dst2, aeT[H:2 * H].reshape(H * EP))

    wl = d1_w1[:, :F]
    wr = d1_w1[:, F:]
    z, logits, u, v = _tc_head(
        aggs2, g2_bias.reshape(1, F), d2_w1, d2_b1.reshape(1, 64),
        d2_w2, d2_b2.reshape(1, 16), wl, wr)

    ps2 = edge_index[0].reshape(E // SUB, SUB)
    pd2 = edge_index[1].reshape(E // SUB, SUB)
    ns2 = neg_edge_index[0].reshape(E // SUB, SUB)
    nd2 = neg_edge_index[1].reshape(E // SUB, SUB)
    cst = jnp.stack([d1_b1, d1_w2[0], jnp.broadcast_to(d1_b2, (64,))])
    pos_pred, neg_pred = _sc_decoder(ps2, pd2, ns2, nd2, u, v, cst)

    all_link_pred = jnp.concatenate([pos_pred, neg_pred], axis=0)
    all_link_label = jnp.concatenate(
        [jnp.ones((E,), jnp.float32), jnp.zeros((E,), jnp.float32)], axis=0)
    return (all_link_pred, all_link_label, logits, z)


# speedup vs baseline: 29.6676x; 1.1345x over previous
import functools

import jax
import jax.numpy as jnp
from jax import lax
from jax.experimental import pallas as pl
from jax.experimental.pallas import tpu as pltpu
from jax.experimental.pallas import tpu_sc as plsc

N = 10000
E = 320000
D = 128
DE = 16
H = 3
HC = 128
F = H * HC
EL = E + N
EP = 330240
NPAD = 10240
DENF = H * NPAD
NCORES = 2
NSUB = 16
NW = NCORES * NSUB
EPW = EP // NW
SUB = 80
ROWS_ALL = EP // SUB
ROWS_W = ROWS_ALL // NW
KA = 688
NCHA = EPW // KA
NSUPB = 3
SUBS_B = ROWS_W // NSUPB
EDGE_B = SUBS_B * SUB
HALFC = 192
EPT_C = E // NW
ROWS_C = EPT_C // SUB
NSUPC = 5
SUBS_C = ROWS_C // NSUPC
EDGE_C = SUBS_C * SUB
NSTRIPE = N // NSUB
ZROWS = 25


def _f32(*shape):
    return jax.ShapeDtypeStruct(shape, jnp.float32)


NB = 400
NBLKS = N // NB


def _proj1_body(x_ref, w_ref, am_ref, xp_ref, a_ref):
    xb = jnp.nan_to_num(x_ref[...], nan=0.0)
    xp = lax.dot_general(xb, w_ref[...], (((1,), (1,)), ((), ())),
                         preferred_element_type=jnp.float32)
    xp_ref[...] = xp
    a_ref[...] = lax.dot_general(xp, am_ref[...], (((1,), (1,)), ((), ())),
                                 preferred_element_type=jnp.float32)


def _tc_proj1(x, w, att_mat):
    return pl.pallas_call(
        _proj1_body,
        grid=(NBLKS,),
        in_specs=[
            pl.BlockSpec((NB, D), lambda i: (i, 0)),
            pl.BlockSpec((F, D), lambda i: (0, 0)),
            pl.BlockSpec((8, F), lambda i: (0, 0)),
        ],
        out_specs=[
            pl.BlockSpec((NB, F), lambda i: (i, 0)),
            pl.BlockSpec((NB, 8), lambda i: (i, 0)),
        ],
        out_shape=[_f32(N, F), _f32(N, 8)],
    )(x, w, att_mat)


EB = 1280
EBLKS = EP // EB


def _edge_att_body(ea_ref, lw1_ref, lw2_ref, m1_ref, m2_ref, o_ref):
    w1t = lax.dot_general(m1_ref[...], lw1_ref[...], (((1,), (0,)), ((), ())),
                          preferred_element_type=jnp.float32)
    w2t = lax.dot_general(m2_ref[...], lw2_ref[...], (((1,), (0,)), ((), ())),
                          preferred_element_type=jnp.float32)
    wcat = jnp.concatenate([w1t, w2t], axis=0)
    ae = lax.dot_general(wcat, ea_ref[...], (((1,), (1,)), ((), ())),
                         preferred_element_type=jnp.float32)
    i = pl.program_id(0)
    col = i * EB + lax.broadcasted_iota(jnp.int32, (6, EB), 1)
    o_ref[...] = jnp.where(col >= EL, -1e30, ae)


def _tc_edge_att(ea_pad, lw1, lw2, ae_mat1, ae_mat2):
    return pl.pallas_call(
        _edge_att_body,
        grid=(EBLKS,),
        in_specs=[
            pl.BlockSpec((EB, DE), lambda i: (i, 0)),
            pl.BlockSpec((F, DE), lambda i: (0, 0)),
            pl.BlockSpec((F, DE), lambda i: (0, 0)),
            pl.BlockSpec((3, F), lambda i: (0, 0)),
            pl.BlockSpec((3, F), lambda i: (0, 0)),
        ],
        out_specs=pl.BlockSpec((6, EB), lambda i: (0, i)),
        out_shape=_f32(6, EP),
    )(ea_pad, lw1, lw2, ae_mat1, ae_mat2)


def _proj2_body(aa_ref, ab_ref, ac_ref, b_ref, w_ref, am_ref, xp_ref, a_ref):
    agg = jnp.concatenate(
        [aa_ref[0] + aa_ref[1], ab_ref[0] + ab_ref[1],
         ac_ref[0] + ac_ref[1]], axis=1) + b_ref[...]
    h = jnp.where(agg > 0, agg, jnp.exp(jnp.minimum(agg, 0.0)) - 1.0)
    xp = lax.dot_general(h, w_ref[...], (((1,), (1,)), ((), ())),
                         preferred_element_type=jnp.float32)
    xp_ref[...] = xp
    a_ref[...] = lax.dot_general(xp, am_ref[...], (((1,), (1,)), ((), ())),
                                 preferred_element_type=jnp.float32)


def _tc_proj2(aggs, bias, w, att_mat):
    return pl.pallas_call(
        _proj2_body,
        grid=(NBLKS,),
        in_specs=[
            pl.BlockSpec((2, NB, HC), lambda i: (0, i, 0)),
            pl.BlockSpec((2, NB, HC), lambda i: (0, i, 0)),
            pl.BlockSpec((2, NB, HC), lambda i: (0, i, 0)),
            pl.BlockSpec((1, F), lambda i: (0, 0)),
            pl.BlockSpec((F, F), lambda i: (0, 0)),
            pl.BlockSpec((8, F), lambda i: (0, 0)),
        ],
        out_specs=[
            pl.BlockSpec((NB, F), lambda i: (i, 0)),
            pl.BlockSpec((NB, 8), lambda i: (i, 0)),
        ],
        out_shape=[_f32(N, F), _f32(N, 8)],
    )(aggs[0], aggs[1], aggs[2], bias, w, att_mat)


def _head_body(aa_ref, ab_ref, ac_ref, b_ref, w1_ref, b1_ref, w2_ref, b2_ref,
               wl_ref, wr_ref, z_ref, lg_ref, u_ref, v_ref):
    z = jnp.concatenate(
        [aa_ref[0] + aa_ref[1], ab_ref[0] + ab_ref[1],
         ac_ref[0] + ac_ref[1]], axis=1) + b_ref[...]
    z_ref[...] = z
    t = lax.dot_general(z, w1_ref[...], (((1,), (1,)), ((), ())),
                        preferred_element_type=jnp.float32) + b1_ref[...]
    t = jnp.maximum(t, 0.0)
    lg = lax.dot_general(t, w2_ref[...], (((1,), (1,)), ((), ())),
                         preferred_element_type=jnp.float32) + b2_ref[...]
    lg_ref[...] = jax.nn.sigmoid(lg)
    u_ref[...] = lax.dot_general(z, wl_ref[...], (((1,), (1,)), ((), ())),
                                 preferred_element_type=jnp.float32)
    v_ref[...] = lax.dot_general(z, wr_ref[...], (((1,), (1,)), ((), ())),
                                 preferred_element_type=jnp.float32)


def _tc_head(aggs, bias, d2w1, d2b1, d2w2, d2b2, wl, wr):
    return pl.pallas_call(
        _head_body,
        grid=(NBLKS,),
        in_specs=[
            pl.BlockSpec((2, NB, HC), lambda i: (0, i, 0)),
            pl.BlockSpec((2, NB, HC), lambda i: (0, i, 0)),
            pl.BlockSpec((2, NB, HC), lambda i: (0, i, 0)),
            pl.BlockSpec((1, F), lambda i: (0, 0)),
            pl.BlockSpec((64, F), lambda i: (0, 0)),
            pl.BlockSpec((1, 64), lambda i: (0, 0)),
            pl.BlockSpec((16, 64), lambda i: (0, 0)),
            pl.BlockSpec((1, 16), lambda i: (0, 0)),
            pl.BlockSpec((64, F), lambda i: (0, 0)),
            pl.BlockSpec((64, F), lambda i: (0, 0)),
        ],
        out_specs=[
            pl.BlockSpec((NB, F), lambda i: (i, 0)),
            pl.BlockSpec((NB, 16), lambda i: (i, 0)),
            pl.BlockSpec((NB, 64), lambda i: (i, 0)),
            pl.BlockSpec((NB, 64), lambda i: (i, 0)),
        ],
        out_shape=[_f32(N, F), _f32(N, 16), _f32(N, 64), _f32(N, 64)],
    )(aggs[0], aggs[1], aggs[2], bias, d2w1, d2b1, d2w2, d2b2, wl, wr)


_MESH = plsc.VectorSubcoreMesh(core_axis_name="c", subcore_axis_name="s")
_SC_PARAMS = pltpu.CompilerParams(use_tc_tiling_on_sc=False,
                                  needs_layout_passes=False)


def _sc_alpha_body(src_hbm, dst_hbm, ae_hbm, atab_hbm, ex_hbm, den_hbm,
                   a_loc, den_loc, srcb, dstb, aeb, exb):
    c = lax.axis_index("c")
    s = lax.axis_index("s")
    wid = s * NCORES + c

    pltpu.sync_copy(atab_hbm, a_loc)

    def zero_body(j, _):
        den_loc[pl.ds(j * 16, 16)] = jnp.zeros((16,), jnp.float32)
        return 0
    lax.fori_loop(0, DENF // 16, zero_body, 0)

    def chunk_body(j, _):
        base = pl.multiple_of(wid * EPW + j * KA, 16)
        pltpu.sync_copy(src_hbm.at[pl.ds(base, KA)], srcb)
        pltpu.sync_copy(dst_hbm.at[pl.ds(base, KA)], dstb)
        for h in range(H):
            pltpu.sync_copy(ae_hbm.at[pl.ds(h * EP + base, KA)], aeb.at[h])

        @plsc.parallel_loop(0, KA // 16, unroll=2)
        def grp_body(g):
            off = g * 16
            si = srcb[pl.ds(off, 16)]
            di = dstb[pl.ds(off, 16)]
            for h in range(H):
                asv = plsc.load_gather(a_loc, [si + h * N])
                adv = plsc.load_gather(a_loc, [di + (H + h) * N])
                al = asv + adv + aeb[h, pl.ds(off, 16)]
                al = jnp.where(al >= 0, al, al * 0.2)
                ev = jnp.exp(al)
                exb[h, pl.ds(off, 16)] = ev
                plsc.addupdate_scatter(den_loc, [di + h * NPAD], ev)

        for h in range(H):
            pltpu.sync_copy(exb.at[h], ex_hbm.at[pl.ds(h * EP + base, KA)])
        return 0
    lax.fori_loop(0, NCHA, chunk_body, 0)

    pltpu.sync_copy(den_loc, den_hbm.at[pl.ds(wid * DENF, DENF)])


def _sc_alpha(src2, dst2, aeT, atab):
    kfn = pl.kernel(
        _sc_alpha_body,
        out_type=(_f32(H * EP), _f32(NW * DENF)),
        mesh=_MESH,
        compiler_params=_SC_PARAMS,
        scratch_types=[
            pltpu.VMEM((8 * N,), jnp.float32),
            pltpu.VMEM((DENF,), jnp.float32),
            pltpu.VMEM((KA,), jnp.int32),
            pltpu.VMEM((KA,), jnp.int32),
            pltpu.VMEM((H, KA), jnp.float32),
            pltpu.VMEM((H, KA), jnp.float32),
        ],
    )
    src_flat = src2.reshape(EP)
    dst_flat = dst2.reshape(EP)
    return kfn(src_flat, dst_flat, aeT, atab)


DBLK = 3840


def _den_red_body(p_ref, o_ref):
    o_ref[...] = jnp.sum(p_ref[...], axis=0, keepdims=True)


def _tc_den_reduce(denp):
    return pl.pallas_call(
        _den_red_body,
        grid=(DENF // DBLK,),
        in_specs=[pl.BlockSpec((NW, DBLK), lambda j: (0, j))],
        out_specs=pl.BlockSpec((1, DBLK), lambda j: (0, j)),
        out_shape=_f32(1, DENF),
    )(denp.reshape(NW, DENF))


def _sc_msg_body(src2_hbm, dst2_hbm, ex_hbm, den_hbm, xph_hbm, out_hbm,
                 den_loc, srcb2, dstb2, exb, coefb, rows, zbuf, acc, sem):
    c = lax.axis_index("c")
    s = lax.axis_index("s")
    wid = s * NCORES + c

    pltpu.sync_copy(den_hbm, den_loc)

    def zzero(j, _):
        def zrow(g, _):
            zbuf[j, pl.ds(g * 16, 16)] = jnp.zeros((16,), jnp.float32)
            return 0
        lax.fori_loop(0, HC // 16, zrow, 0)
        return 0
    lax.fori_loop(0, ZROWS, zzero, 0)

    def zcopy(k, _):
        pltpu.sync_copy(
            zbuf, acc.at[pl.ds(s * NSTRIPE + k * ZROWS, ZROWS), :])
        return 0
    lax.fori_loop(0, NSTRIPE // ZROWS, zcopy, 0)
    plsc.subcore_barrier()

    for sup in range(NSUPB):
        rowbase = wid * ROWS_W + sup * SUBS_B
        ebase = rowbase * SUB
        pltpu.sync_copy(src2_hbm.at[pl.ds(rowbase, SUBS_B), :], srcb2)
        pltpu.sync_copy(dst2_hbm.at[pl.ds(rowbase, SUBS_B), :], dstb2)
        pltpu.sync_copy(ex_hbm.at[pl.ds(ebase, EDGE_B)], exb)

        @plsc.parallel_loop(0, EDGE_B // 16, unroll=2)
        def coef_body(g):
            sub = g // (SUB // 16)
            off = (g % (SUB // 16)) * 16
            di = dstb2[sub, pl.ds(off, 16)]
            fl = pl.ds(sub * SUB + off, 16)
            denv = plsc.load_gather(den_loc, [di])
            coefb[fl] = exb[fl] / denv

        def sub_body(k, _):
            pltpu.async_copy(xph_hbm.at[srcb2.at[k]], rows, sem).wait()

            @plsc.parallel_loop(0, SUB, unroll=4)
            def scale_body(e):
                csv = plsc.load_gather(
                    coefb, [jnp.full((16,), k * SUB + e, jnp.int32)])
                for grp in range(HC // 16):
                    sl = pl.ds(grp * 16, 16)
                    rows[e, sl] = rows[e, sl] * csv
            pltpu.sync_copy(rows, acc.at[dstb2.at[k]], add=True)
            return 0
        lax.fori_loop(0, SUBS_B, sub_body, 0)

    plsc.subcore_barrier()
    ro = s * NSTRIPE
    pltpu.sync_copy(acc.at[pl.ds(ro, NSTRIPE), :],
                    out_hbm.at[c, pl.ds(ro, NSTRIPE), :])


def _sc_msg(src2, dst2, exh, denh, xph):
    kfn = pl.kernel(
        _sc_msg_body,
        out_type=_f32(NCORES, N, HC),
        mesh=_MESH,
        compiler_params=_SC_PARAMS,
        scratch_types=[
            pltpu.VMEM((NPAD,), jnp.float32),
            pltpu.VMEM((SUBS_B, SUB), jnp.int32),
            pltpu.VMEM((SUBS_B, SUB), jnp.int32),
            pltpu.VMEM((EDGE_B,), jnp.float32),
            pltpu.VMEM((EDGE_B,), jnp.float32),
            pltpu.VMEM((SUB, HC), jnp.float32),
            pltpu.VMEM((ZROWS, HC), jnp.float32),
            pltpu.VMEM_SHARED((N, HC), jnp.float32),
            pltpu.SemaphoreType.DMA,
        ],
    )
    return kfn(src2, dst2, exh, denh, xph)


def _sc_decoder_body(ps2_hbm, pd2_hbm, ns2_hbm, nd2_hbm, u_hbm, v_hbm,
                     cst_hbm, pos_hbm, neg_hbm,
                     srcb2, dstb2, urows, vrows, predb, cstb, sem):
    c = lax.axis_index("c")
    s = lax.axis_index("s")
    wid = s * NCORES + c
    pltpu.sync_copy(cst_hbm, cstb)
    b2s = cstb[2, pl.ds(0, 16)][0]

    for (s2, d2, o_hbm) in ((ps2_hbm, pd2_hbm, pos_hbm),
                            (ns2_hbm, nd2_hbm, neg_hbm)):
        for sup in range(NSUPC):
            rowbase = wid * ROWS_C + sup * SUBS_C
            ebase = rowbase * SUB
            pltpu.sync_copy(s2.at[pl.ds(rowbase, SUBS_C), :], srcb2)
            pltpu.sync_copy(d2.at[pl.ds(rowbase, SUBS_C), :], dstb2)

            def sub_body(k, _):
                pltpu.async_copy(u_hbm.at[srcb2.at[k]], urows, sem).wait()
                pltpu.async_copy(v_hbm.at[dstb2.at[k]], vrows, sem).wait()
                lanes = lax.iota(jnp.int32, 16)

                def grp_body(g, _):
                    @plsc.parallel_loop(
                        0, 16, unroll=4,
                        carry=jnp.zeros((16,), jnp.float32))
                    def edge_body(i, resv):
                        e = g * 16 + i
                        accv = jnp.zeros((16,), jnp.float32)
                        for grp in range(4):
                            sl = pl.ds(grp * 16, 16)
                            t = urows[e, sl] + vrows[e, sl] + cstb[0, sl]
                            t = jnp.maximum(t, 0.0)
                            accv = accv + t * cstb[1, sl]
                        tot = jnp.sum(accv) + b2s
                        return jnp.where(lanes == i, tot, resv)
                    predb[pl.ds(k * SUB + g * 16, 16)] = edge_body
                    return 0
                lax.fori_loop(0, SUB // 16, grp_body, 0)
                return 0
            lax.fori_loop(0, SUBS_C, sub_body, 0)
            pltpu.sync_copy(predb, o_hbm.at[pl.ds(ebase, EDGE_C)])


def _sc_decoder(ps2, pd2, ns2, nd2, u, v, cst):
    kfn = pl.kernel(
        _sc_decoder_body,
        out_type=(_f32(E), _f32(E)),
        mesh=_MESH,
        compiler_params=_SC_PARAMS,
        scratch_types=[
            pltpu.VMEM((SUBS_C, SUB), jnp.int32),
            pltpu.VMEM((SUBS_C, SUB), jnp.int32),
            pltpu.VMEM((SUB, 64), jnp.float32),
            pltpu.VMEM((SUB, 64), jnp.float32),
            pltpu.VMEM((EDGE_C,), jnp.float32),
            pltpu.VMEM((3, 64), jnp.float32),
            pltpu.SemaphoreType.DMA,
        ],
    )
    return kfn(ps2, pd2, ns2, nd2, u, v, cst)


def _place_heads(att):
    rows = []
    for h in range(H):
        parts = [jnp.zeros((HC,), jnp.float32)] * H
        parts[h] = att[0, h]
        rows.append(jnp.concatenate(parts))
    return jnp.stack(rows)


def _att_mat(att_src, att_dst):
    return jnp.concatenate(
        [_place_heads(att_src), _place_heads(att_dst),
         jnp.zeros((2, F), jnp.float32)], axis=0)


def _gat_layer(xp, a_tab, src2, dst2, aeT3):
    exT, denp = _sc_alpha(src2, dst2, aeT3, a_tab.T.reshape(8 * N))
    den = _tc_den_reduce(denp).reshape(DENF)
    xp3 = xp.reshape(N, H, HC).transpose(1, 0, 2)
    return tuple(
        _sc_msg(src2, dst2, exT[h * EP:(h + 1) * EP],
                den[h * NPAD:(h + 1) * NPAD], xp3[h])
        for h in range(H))


def kernel(x, edge_index, edge_attr, neg_edge_index, g1_lin_w, g1_att_src,
           g1_att_dst, g1_lin_edge_w, g1_att_edge, g1_bias, g2_lin_w,
           g2_att_src, g2_att_dst, g2_lin_edge_w, g2_att_edge, g2_bias,
           d1_w1, d1_b1, d1_w2, d1_b2, d2_w1, d2_b1, d2_w2, d2_b2):
    loop = jnp.arange(N, dtype=jnp.int32)
    padi = jnp.zeros((EP - EL,), jnp.int32)
    src2 = jnp.concatenate([edge_index[0], loop, padi]).reshape(ROWS_ALL, SUB)
    dst2 = jnp.concatenate([edge_index[1], loop, padi]).reshape(ROWS_ALL, SUB)
    ea_pad = jnp.concatenate([
        edge_attr, jnp.ones((N, DE), jnp.float32),
        jnp.zeros((EP - EL, DE), jnp.float32)], axis=0)

    att_mat1 = _att_mat(g1_att_src, g1_att_dst)
    att_mat2 = _att_mat(g2_att_src, g2_att_dst)
    ae_mat1 = _place_heads(g1_att_edge)
    ae_mat2 = _place_heads(g2_att_edge)

    xp1, a1 = _tc_proj1(x, g1_lin_w, att_mat1)
    aeT = _tc_edge_att(ea_pad, g1_lin_edge_w, g2_lin_edge_w, ae_mat1, ae_mat2)

    aggs1 = _gat_layer(xp1, a1, src2, dst2, aeT[0:H].reshape(H * EP))

    xp2, a2 = _tc_proj2(aggs1, g1_bias.reshape(1, F), g2_lin_w, att_mat2)

    aggs2 = _gat_layer(xp2, a2, src2, dst2, aeT[H:2 * H].reshape(H * EP))

    wl = d1_w1[:, :F]
    wr = d1_w1[:, F:]
    z, logits, u, v = _tc_head(
        aggs2, g2_bias.reshape(1, F), d2_w1, d2_b1.reshape(1, 64),
        d2_w2, d2_b2.reshape(1, 16), wl, wr)

    ps2 = edge_index[0].reshape(E // SUB, SUB)
    pd2 = edge_index[1].reshape(E // SUB, SUB)
    ns2 = neg_edge_index[0].reshape(E // SUB, SUB)
    nd2 = neg_edge_index[1].reshape(E // SUB, SUB)
    cst = jnp.stack([d1_b1, d1_w2[0], jnp.broadcast_to(d1_b2, (64,))])
    pos_pred, neg_pred = _sc_decoder(ps2, pd2, ns2, nd2, u, v, cst)

    all_link_pred = jnp.concatenate([pos_pred, neg_pred], axis=0)
    all_link_label = jnp.concatenate(
        [jnp.ones((E,), jnp.float32), jnp.zeros((E,), jnp.float32)], axis=0)
    return (all_link_pred, all_link_label, logits, z)
